# baseline (speedup 1.0000x reference)
import functools

import jax
import jax.numpy as jnp
from jax import lax
from jax.experimental import pallas as pl
from jax.experimental.pallas import tpu as pltpu
from jax.experimental.pallas import tpu_sc as plsc

NUM_CLUSTERS = 8
HIDDEN_DIM = 2048
BOTTLENECK_DIM = 512
N_TOKENS = 4096
ROW_BLOCK = 256
M_TILES = N_TOKENS // ROW_BLOCK
GRID_T = M_TILES + NUM_CLUSTERS - 1


def _route_metadata(cid):
    onehot = (cid[:, None] == jnp.arange(NUM_CLUSTERS)[None, :]).astype(
        jnp.int32)
    cum = jnp.cumsum(onehot, axis=0)
    counts = cum[-1]
    off = jnp.concatenate(
        [jnp.zeros((1,), jnp.int32), jnp.cumsum(counts).astype(jnp.int32)])
    rank = off[cid] + jnp.take_along_axis(cum, cid[:, None], axis=1)[:, 0] - 1
    start_tile = off[:-1] // ROW_BLOCK
    end_tile = jnp.where(counts > 0,
                         (off[1:] + ROW_BLOCK - 1) // ROW_BLOCK, start_tile)
    ntiles = end_tile - start_tile
    cum_t = jnp.cumsum(ntiles)
    t = jnp.arange(GRID_T, dtype=jnp.int32)
    g = jnp.searchsorted(cum_t, t, side="right").astype(jnp.int32)
    g = jnp.minimum(g, NUM_CLUSTERS - 1)
    prev = jnp.where(g > 0, cum_t[jnp.maximum(g - 1, 0)], 0).astype(jnp.int32)
    tile = jnp.clip(start_tile[g] + (t - prev), 0, M_TILES - 1)
    return rank, off, g, tile


_SC_CORES = 2
_SC_SUBCORES = 16
_SC_WORKERS = _SC_CORES * _SC_SUBCORES
_SC_CHUNK = 16
_SC_NCHUNK = N_TOKENS // _SC_WORKERS // _SC_CHUNK
_SC_ROWS_PER_W = _SC_NCHUNK * _SC_CHUNK


def _sc_scatter_rows(h, rank):
    rank3 = rank.reshape(_SC_WORKERS, _SC_NCHUNK, _SC_CHUNK)
    mesh = plsc.VectorSubcoreMesh(core_axis_name="c", subcore_axis_name="s")

    @functools.partial(
        pl.kernel,
        out_type=jax.ShapeDtypeStruct((N_TOKENS, HIDDEN_DIM), jnp.float32),
        mesh=mesh,
        scratch_types=[
            pltpu.VMEM((_SC_NCHUNK, _SC_CHUNK), jnp.int32),
            pltpu.VMEM((2, _SC_CHUNK, HIDDEN_DIM), jnp.float32),
            pltpu.SemaphoreType.DMA,
            pltpu.SemaphoreType.DMA,
        ],
    )
    def k(h_hbm, rank_hbm, out_hbm, idx_v, rows_v, sem_in, sem_out):
        wid = lax.axis_index("s") * _SC_CORES + lax.axis_index("c")
        pltpu.sync_copy(rank_hbm.at[wid], idx_v)
        base = wid * _SC_ROWS_PER_W
        reads = [None, None]
        scats = [None, None]
        reads[0] = pltpu.async_copy(
            h_hbm.at[pl.ds(base, _SC_CHUNK)], rows_v.at[0], sem_in)
        for c in range(_SC_NCHUNK):
            b = c % 2
            nb = (c + 1) % 2
            if c + 1 < _SC_NCHUNK:
                if scats[nb] is not None:
                    scats[nb].wait()
                reads[nb] = pltpu.async_copy(
                    h_hbm.at[pl.ds(base + (c + 1) * _SC_CHUNK, _SC_CHUNK)],
                    rows_v.at[nb], sem_in)
            reads[b].wait()
            scats[b] = pltpu.async_copy(
                rows_v.at[b], out_hbm.at[idx_v.at[c]], sem_out)
        scats[(_SC_NCHUNK - 1) % 2].wait()
        scats[_SC_NCHUNK % 2].wait()

    return k(h, rank3)


def _gmm_body(g_ref, tile_ref, off_ref, hs_ref, w1_ref, b1_ref, w2_ref,
              b2_ref, out_ref):
    t = pl.program_id(0)
    g = g_ref[t]
    tile = tile_ref[t]
    first = jnp.logical_or(t == 0, tile != tile_ref[jnp.maximum(t - 1, 0)])

    @pl.when(first)
    def _init():
        out_ref[...] = jnp.zeros_like(out_ref)

    x = hs_ref[...]
    z = lax.dot_general(x.astype(jnp.bfloat16), w1_ref[0].astype(jnp.bfloat16),
                        (((1,), (1,)), ((), ())),
                        preferred_element_type=jnp.float32)
    z = z + b1_ref[0]
    z = 0.5 * z * (1.0 + lax.erf(z * 0.7071067811865476))
    delta = lax.dot_general(z.astype(jnp.bfloat16),
                            w2_ref[0].astype(jnp.bfloat16),
                            (((1,), (1,)), ((), ())),
                            preferred_element_type=jnp.float32)
    out = x + (delta + b2_ref[0])
    row = lax.broadcasted_iota(jnp.int32, (ROW_BLOCK, 1), 0) + tile * ROW_BLOCK
    mask = jnp.logical_and(row >= off_ref[g], row < off_ref[g + 1])
    out_ref[...] = jnp.where(mask, out, out_ref[...])


def _grouped_delta(hs, W1, b1r, W2, b2r, off, g, tile):
    grid_spec = pltpu.PrefetchScalarGridSpec(
        num_scalar_prefetch=3,
        grid=(GRID_T,),
        in_specs=[
            pl.BlockSpec((ROW_BLOCK, HIDDEN_DIM),
                         lambda t, gr, tr, orf: (tr[t], 0)),
            pl.BlockSpec((1, BOTTLENECK_DIM, HIDDEN_DIM),
                         lambda t, gr, tr, orf: (gr[t], 0, 0)),
            pl.BlockSpec((1, 1, BOTTLENECK_DIM),
                         lambda t, gr, tr, orf: (gr[t], 0, 0)),
            pl.BlockSpec((1, HIDDEN_DIM, BOTTLENECK_DIM),
                         lambda t, gr, tr, orf: (gr[t], 0, 0)),
            pl.BlockSpec((1, 1, HIDDEN_DIM),
                         lambda t, gr, tr, orf: (gr[t], 0, 0)),
        ],
        out_specs=pl.BlockSpec((ROW_BLOCK, HIDDEN_DIM),
                               lambda t, gr, tr, orf: (tr[t], 0)),
    )
    return pl.pallas_call(
        _gmm_body,
        grid_spec=grid_spec,
        out_shape=jax.ShapeDtypeStruct((N_TOKENS, HIDDEN_DIM), jnp.float32),
        compiler_params=pltpu.CompilerParams(
            dimension_semantics=("arbitrary",)),
    )(g, tile, off, hs, W1, b1r, W2, b2r)


def kernel(h, cluster_ids, W1, b1, W2, b2):
    cid = cluster_ids.astype(jnp.int32)
    rank, off, g, tile = _route_metadata(cid)

    hs = _sc_scatter_rows(h, rank)
    b1r = b1.reshape(NUM_CLUSTERS, 1, BOTTLENECK_DIM)
    b2r = b2.reshape(NUM_CLUSTERS, 1, HIDDEN_DIM)

    out_sorted = _grouped_delta(hs, W1, b1r, W2, b2r, off, g, tile)
    return jnp.take(out_sorted, rank, axis=0)


# speedup vs baseline: 1.0268x; 1.0268x over previous
import functools

import jax
import jax.numpy as jnp
from jax import lax
from jax.experimental import pallas as pl
from jax.experimental.pallas import tpu as pltpu
from jax.experimental.pallas import tpu_sc as plsc

NUM_CLUSTERS = 8
HIDDEN_DIM = 2048
BOTTLENECK_DIM = 512
N_TOKENS = 4096
ROW_BLOCK = 512
M_TILES = N_TOKENS // ROW_BLOCK
GRID_T = M_TILES + NUM_CLUSTERS - 1


def _route_metadata(cid):
    onehot = (cid[:, None] == jnp.arange(NUM_CLUSTERS)[None, :]).astype(
        jnp.int32)
    cum = jnp.cumsum(onehot, axis=0)
    counts = cum[-1]
    off = jnp.concatenate(
        [jnp.zeros((1,), jnp.int32), jnp.cumsum(counts).astype(jnp.int32)])
    rank = off[cid] + jnp.take_along_axis(cum, cid[:, None], axis=1)[:, 0] - 1
    start_tile = off[:-1] // ROW_BLOCK
    end_tile = jnp.where(counts > 0,
                         (off[1:] + ROW_BLOCK - 1) // ROW_BLOCK, start_tile)
    ntiles = end_tile - start_tile
    cum_t = jnp.cumsum(ntiles)
    t = jnp.arange(GRID_T, dtype=jnp.int32)
    g = jnp.searchsorted(cum_t, t, side="right").astype(jnp.int32)
    g = jnp.minimum(g, NUM_CLUSTERS - 1)
    prev = jnp.where(g > 0, cum_t[jnp.maximum(g - 1, 0)], 0).astype(jnp.int32)
    tile = jnp.clip(start_tile[g] + (t - prev), 0, M_TILES - 1)
    return rank, off, g, tile


_SC_CORES = 2
_SC_SUBCORES = 16
_SC_WORKERS = _SC_CORES * _SC_SUBCORES
_SC_CHUNK = 16
_SC_NCHUNK = N_TOKENS // _SC_WORKERS // _SC_CHUNK
_SC_ROWS_PER_W = _SC_NCHUNK * _SC_CHUNK


def _sc_scatter_rows(h, rank):
    rank3 = rank.reshape(_SC_WORKERS, _SC_NCHUNK, _SC_CHUNK)
    mesh = plsc.VectorSubcoreMesh(core_axis_name="c", subcore_axis_name="s")

    @functools.partial(
        pl.kernel,
        out_type=jax.ShapeDtypeStruct((N_TOKENS, HIDDEN_DIM), jnp.float32),
        mesh=mesh,
        scratch_types=[
            pltpu.VMEM((_SC_NCHUNK, _SC_CHUNK), jnp.int32),
            pltpu.VMEM((2, _SC_CHUNK, HIDDEN_DIM), jnp.float32),
            pltpu.SemaphoreType.DMA,
            pltpu.SemaphoreType.DMA,
        ],
    )
    def k(h_hbm, rank_hbm, out_hbm, idx_v, rows_v, sem_in, sem_out):
        wid = lax.axis_index("s") * _SC_CORES + lax.axis_index("c")
        pltpu.sync_copy(rank_hbm.at[wid], idx_v)
        base = wid * _SC_ROWS_PER_W
        reads = [None, None]
        scats = [None, None]
        reads[0] = pltpu.async_copy(
            h_hbm.at[pl.ds(base, _SC_CHUNK)], rows_v.at[0], sem_in)
        for c in range(_SC_NCHUNK):
            b = c % 2
            nb = (c + 1) % 2
            if c + 1 < _SC_NCHUNK:
                if scats[nb] is not None:
                    scats[nb].wait()
                reads[nb] = pltpu.async_copy(
                    h_hbm.at[pl.ds(base + (c + 1) * _SC_CHUNK, _SC_CHUNK)],
                    rows_v.at[nb], sem_in)
            reads[b].wait()
            scats[b] = pltpu.async_copy(
                rows_v.at[b], out_hbm.at[idx_v.at[c]], sem_out)
        scats[(_SC_NCHUNK - 1) % 2].wait()
        scats[_SC_NCHUNK % 2].wait()

    return k(h, rank3)


def _gmm_body(g_ref, tile_ref, off_ref, hs_ref, w1_ref, b1_ref, w2_ref,
              b2_ref, out_ref):
    t = pl.program_id(0)
    g = g_ref[t]
    tile = tile_ref[t]
    first = jnp.logical_or(t == 0, tile != tile_ref[jnp.maximum(t - 1, 0)])

    @pl.when(first)
    def _init():
        out_ref[...] = jnp.zeros_like(out_ref)

    x = hs_ref[...]
    z = lax.dot_general(x.astype(jnp.bfloat16), w1_ref[0].astype(jnp.bfloat16),
                        (((1,), (1,)), ((), ())),
                        preferred_element_type=jnp.float32)
    z = z + b1_ref[0]
    z = 0.5 * z * (1.0 + lax.erf(z * 0.7071067811865476))
    delta = lax.dot_general(z.astype(jnp.bfloat16),
                            w2_ref[0].astype(jnp.bfloat16),
                            (((1,), (1,)), ((), ())),
                            preferred_element_type=jnp.float32)
    out = x + (delta + b2_ref[0])
    row = lax.broadcasted_iota(jnp.int32, (ROW_BLOCK, 1), 0) + tile * ROW_BLOCK
    mask = jnp.logical_and(row >= off_ref[g], row < off_ref[g + 1])
    out_ref[...] = jnp.where(mask, out, out_ref[...])


def _grouped_delta(hs, W1, b1r, W2, b2r, off, g, tile):
    grid_spec = pltpu.PrefetchScalarGridSpec(
        num_scalar_prefetch=3,
        grid=(GRID_T,),
        in_specs=[
            pl.BlockSpec((ROW_BLOCK, HIDDEN_DIM),
                         lambda t, gr, tr, orf: (tr[t], 0)),
            pl.BlockSpec((1, BOTTLENECK_DIM, HIDDEN_DIM),
                         lambda t, gr, tr, orf: (gr[t], 0, 0)),
            pl.BlockSpec((1, 1, BOTTLENECK_DIM),
                         lambda t, gr, tr, orf: (gr[t], 0, 0)),
            pl.BlockSpec((1, HIDDEN_DIM, BOTTLENECK_DIM),
                         lambda t, gr, tr, orf: (gr[t], 0, 0)),
            pl.BlockSpec((1, 1, HIDDEN_DIM),
                         lambda t, gr, tr, orf: (gr[t], 0, 0)),
        ],
        out_specs=pl.BlockSpec((ROW_BLOCK, HIDDEN_DIM),
                               lambda t, gr, tr, orf: (tr[t], 0)),
    )
    return pl.pallas_call(
        _gmm_body,
        grid_spec=grid_spec,
        out_shape=jax.ShapeDtypeStruct((N_TOKENS, HIDDEN_DIM), jnp.float32),
        compiler_params=pltpu.CompilerParams(
            dimension_semantics=("arbitrary",)),
    )(g, tile, off, hs, W1, b1r, W2, b2r)


def kernel(h, cluster_ids, W1, b1, W2, b2):
    cid = cluster_ids.astype(jnp.int32)
    rank, off, g, tile = _route_metadata(cid)

    hs = _sc_scatter_rows(h, rank)
    b1r = b1.reshape(NUM_CLUSTERS, 1, BOTTLENECK_DIM)
    b2r = b2.reshape(NUM_CLUSTERS, 1, HIDDEN_DIM)

    out_sorted = _grouped_delta(hs, W1, b1r, W2, b2r, off, g, tile)
    return jnp.take(out_sorted, rank, axis=0)


# speedup vs baseline: 1.1894x; 1.1584x over previous
import functools

import jax
import jax.numpy as jnp
from jax import lax
from jax.experimental import pallas as pl
from jax.experimental.pallas import tpu as pltpu
from jax.experimental.pallas import tpu_sc as plsc

NUM_CLUSTERS = 8
HIDDEN_DIM = 2048
BOTTLENECK_DIM = 512
N_TOKENS = 4096
ROW_BLOCK = 512
M_TILES = N_TOKENS // ROW_BLOCK
GRID_T = M_TILES + NUM_CLUSTERS - 1


def _route_metadata(cid):
    onehot = (cid[:, None] == jnp.arange(NUM_CLUSTERS)[None, :]).astype(
        jnp.int32)
    cum = jnp.cumsum(onehot, axis=0)
    counts = cum[-1]
    off = jnp.concatenate(
        [jnp.zeros((1,), jnp.int32), jnp.cumsum(counts).astype(jnp.int32)])
    rank = off[cid] + jnp.take_along_axis(cum, cid[:, None], axis=1)[:, 0] - 1
    start_tile = off[:-1] // ROW_BLOCK
    end_tile = jnp.where(counts > 0,
                         (off[1:] + ROW_BLOCK - 1) // ROW_BLOCK, start_tile)
    ntiles = end_tile - start_tile
    cum_t = jnp.cumsum(ntiles)
    t = jnp.arange(GRID_T, dtype=jnp.int32)
    g = jnp.searchsorted(cum_t, t, side="right").astype(jnp.int32)
    g = jnp.minimum(g, NUM_CLUSTERS - 1)
    prev = jnp.where(g > 0, cum_t[jnp.maximum(g - 1, 0)], 0).astype(jnp.int32)
    tile = jnp.clip(start_tile[g] + (t - prev), 0, M_TILES - 1)
    return rank, off, g, tile


_SC_CORES = 2
_SC_SUBCORES = 16
_SC_WORKERS = _SC_CORES * _SC_SUBCORES
_SC_CHUNK = 16
_SC_NCHUNK = N_TOKENS // _SC_WORKERS // _SC_CHUNK
_SC_ROWS_PER_W = _SC_NCHUNK * _SC_CHUNK


def _sc_scatter_rows(h, rank):
    rank3 = rank.reshape(_SC_WORKERS, _SC_NCHUNK, _SC_CHUNK)
    mesh = plsc.VectorSubcoreMesh(core_axis_name="c", subcore_axis_name="s")

    @functools.partial(
        pl.kernel,
        out_type=jax.ShapeDtypeStruct((N_TOKENS, HIDDEN_DIM), jnp.float32),
        mesh=mesh,
        scratch_types=[
            pltpu.VMEM((_SC_NCHUNK, _SC_CHUNK), jnp.int32),
            pltpu.VMEM((2, _SC_CHUNK, HIDDEN_DIM), jnp.float32),
            pltpu.SemaphoreType.DMA,
            pltpu.SemaphoreType.DMA,
        ],
    )
    def k(h_hbm, rank_hbm, out_hbm, idx_v, rows_v, sem_in, sem_out):
        wid = lax.axis_index("s") * _SC_CORES + lax.axis_index("c")
        pltpu.sync_copy(rank_hbm.at[wid], idx_v)
        base = wid * _SC_ROWS_PER_W
        reads = [None, None]
        scats = [None, None]
        reads[0] = pltpu.async_copy(
            h_hbm.at[pl.ds(base, _SC_CHUNK)], rows_v.at[0], sem_in)
        for c in range(_SC_NCHUNK):
            b = c % 2
            nb = (c + 1) % 2
            if c + 1 < _SC_NCHUNK:
                if scats[nb] is not None:
                    scats[nb].wait()
                reads[nb] = pltpu.async_copy(
                    h_hbm.at[pl.ds(base + (c + 1) * _SC_CHUNK, _SC_CHUNK)],
                    rows_v.at[nb], sem_in)
            reads[b].wait()
            scats[b] = pltpu.async_copy(
                rows_v.at[b], out_hbm.at[idx_v.at[c]], sem_out)
        scats[(_SC_NCHUNK - 1) % 2].wait()
        scats[_SC_NCHUNK % 2].wait()

    return k(h, rank3)


def _gmm_body(g_ref, tile_ref, off_ref, hs_ref, w1_ref, b1_ref, w2_ref,
              b2_ref, out_ref):
    t = pl.program_id(0)
    g = g_ref[t]
    tile = tile_ref[t]
    first = jnp.logical_or(t == 0, tile != tile_ref[jnp.maximum(t - 1, 0)])

    @pl.when(first)
    def _init():
        out_ref[...] = jnp.zeros_like(out_ref)

    x = hs_ref[...]
    z = lax.dot_general(x.astype(jnp.bfloat16), w1_ref[0].astype(jnp.bfloat16),
                        (((1,), (1,)), ((), ())),
                        preferred_element_type=jnp.float32)
    z = z + b1_ref[0]
    z = 0.5 * z * (1.0 + lax.erf(z * 0.7071067811865476))
    delta = lax.dot_general(z.astype(jnp.bfloat16),
                            w2_ref[0].astype(jnp.bfloat16),
                            (((1,), (1,)), ((), ())),
                            preferred_element_type=jnp.float32)
    out = x + (delta + b2_ref[0])
    row = lax.broadcasted_iota(jnp.int32, (ROW_BLOCK, 1), 0) + tile * ROW_BLOCK
    mask = jnp.logical_and(row >= off_ref[g], row < off_ref[g + 1])
    out_ref[...] = jnp.where(mask, out, out_ref[...])


def _grouped_delta(hs, W1, b1r, W2, b2r, off, g, tile):
    grid_spec = pltpu.PrefetchScalarGridSpec(
        num_scalar_prefetch=3,
        grid=(GRID_T,),
        in_specs=[
            pl.BlockSpec((ROW_BLOCK, HIDDEN_DIM),
                         lambda t, gr, tr, orf: (tr[t], 0)),
            pl.BlockSpec((1, BOTTLENECK_DIM, HIDDEN_DIM),
                         lambda t, gr, tr, orf: (gr[t], 0, 0)),
            pl.BlockSpec((1, 1, BOTTLENECK_DIM),
                         lambda t, gr, tr, orf: (gr[t], 0, 0)),
            pl.BlockSpec((1, HIDDEN_DIM, BOTTLENECK_DIM),
                         lambda t, gr, tr, orf: (gr[t], 0, 0)),
            pl.BlockSpec((1, 1, HIDDEN_DIM),
                         lambda t, gr, tr, orf: (gr[t], 0, 0)),
        ],
        out_specs=pl.BlockSpec((ROW_BLOCK, HIDDEN_DIM),
                               lambda t, gr, tr, orf: (tr[t], 0)),
    )
    return pl.pallas_call(
        _gmm_body,
        grid_spec=grid_spec,
        out_shape=jax.ShapeDtypeStruct((N_TOKENS, HIDDEN_DIM), jnp.float32),
        compiler_params=pltpu.CompilerParams(
            dimension_semantics=("arbitrary",)),
    )(g, tile, off, hs, W1, b1r, W2, b2r)


def kernel(h, cluster_ids, W1, b1, W2, b2):
    cid = cluster_ids.astype(jnp.int32)
    rank, off, g, tile = _route_metadata(cid)

    hs = _sc_scatter_rows(h, rank)
    b1r = b1.reshape(NUM_CLUSTERS, 1, BOTTLENECK_DIM)
    b2r = b2.reshape(NUM_CLUSTERS, 1, HIDDEN_DIM)

    out_sorted = _grouped_delta(hs, W1, b1r, W2, b2r, off, g, tile)
    return jnp.take(out_sorted, rank, axis=0, mode="clip")


# speedup vs baseline: 1.1969x; 1.0062x over previous
import functools

import jax
import jax.numpy as jnp
from jax import lax
from jax.experimental import pallas as pl
from jax.experimental.pallas import tpu as pltpu
from jax.experimental.pallas import tpu_sc as plsc

NUM_CLUSTERS = 8
HIDDEN_DIM = 2048
BOTTLENECK_DIM = 512
N_TOKENS = 4096
ROW_BLOCK = 512
M_TILES = N_TOKENS // ROW_BLOCK
GRID_T = M_TILES + NUM_CLUSTERS - 1


def _route_metadata(cid):
    ar = jnp.arange(NUM_CLUSTERS, dtype=jnp.int32)
    onehot = (cid[None, :] == ar[:, None]).astype(jnp.int32)
    cum = jnp.cumsum(onehot, axis=1)
    counts = cum[:, -1]
    off = jnp.concatenate(
        [jnp.zeros((1,), jnp.int32), jnp.cumsum(counts).astype(jnp.int32)])
    rank = off[cid] + cum[cid, jnp.arange(N_TOKENS)] - 1
    start_tile = off[:-1] // ROW_BLOCK
    end_tile = jnp.where(counts > 0,
                         (off[1:] + ROW_BLOCK - 1) // ROW_BLOCK, start_tile)
    ntiles = end_tile - start_tile
    cum_t = jnp.cumsum(ntiles)
    t = jnp.arange(GRID_T, dtype=jnp.int32)
    g = jnp.sum((cum_t[None, :] <= t[:, None]).astype(jnp.int32), axis=1)
    g = jnp.minimum(g, NUM_CLUSTERS - 1)
    prev = jnp.where(g > 0, cum_t[jnp.maximum(g - 1, 0)], 0).astype(jnp.int32)
    tile = jnp.clip(start_tile[g] + (t - prev), 0, M_TILES - 1)
    return rank, off, g, tile


_SC_CORES = 2
_SC_SUBCORES = 16
_SC_WORKERS = _SC_CORES * _SC_SUBCORES
_SC_CHUNK = 16
_SC_NCHUNK = N_TOKENS // _SC_WORKERS // _SC_CHUNK
_SC_ROWS_PER_W = _SC_NCHUNK * _SC_CHUNK


def _sc_scatter_rows(h, rank):
    rank3 = rank.reshape(_SC_WORKERS, _SC_NCHUNK, _SC_CHUNK)
    mesh = plsc.VectorSubcoreMesh(core_axis_name="c", subcore_axis_name="s")

    @functools.partial(
        pl.kernel,
        out_type=jax.ShapeDtypeStruct((N_TOKENS, HIDDEN_DIM), jnp.float32),
        mesh=mesh,
        scratch_types=[
            pltpu.VMEM((_SC_NCHUNK, _SC_CHUNK), jnp.int32),
            pltpu.VMEM((2, _SC_CHUNK, HIDDEN_DIM), jnp.float32),
            pltpu.SemaphoreType.DMA,
            pltpu.SemaphoreType.DMA,
        ],
    )
    def k(h_hbm, rank_hbm, out_hbm, idx_v, rows_v, sem_in, sem_out):
        wid = lax.axis_index("s") * _SC_CORES + lax.axis_index("c")
        pltpu.sync_copy(rank_hbm.at[wid], idx_v)
        base = wid * _SC_ROWS_PER_W
        reads = [None, None]
        scats = [None, None]
        reads[0] = pltpu.async_copy(
            h_hbm.at[pl.ds(base, _SC_CHUNK)], rows_v.at[0], sem_in)
        for c in range(_SC_NCHUNK):
            b = c % 2
            nb = (c + 1) % 2
            if c + 1 < _SC_NCHUNK:
                if scats[nb] is not None:
                    scats[nb].wait()
                reads[nb] = pltpu.async_copy(
                    h_hbm.at[pl.ds(base + (c + 1) * _SC_CHUNK, _SC_CHUNK)],
                    rows_v.at[nb], sem_in)
            reads[b].wait()
            scats[b] = pltpu.async_copy(
                rows_v.at[b], out_hbm.at[idx_v.at[c]], sem_out)
        scats[(_SC_NCHUNK - 1) % 2].wait()
        scats[_SC_NCHUNK % 2].wait()

    return k(h, rank3)


def _gmm_body(g_ref, tile_ref, off_ref, hs_ref, w1_ref, b1_ref, w2_ref,
              b2_ref, out_ref):
    t = pl.program_id(0)
    g = g_ref[t]
    tile = tile_ref[t]
    first = jnp.logical_or(t == 0, tile != tile_ref[jnp.maximum(t - 1, 0)])

    @pl.when(first)
    def _init():
        out_ref[...] = jnp.zeros_like(out_ref)

    x = hs_ref[...]
    z = lax.dot_general(x.astype(jnp.bfloat16), w1_ref[0].astype(jnp.bfloat16),
                        (((1,), (1,)), ((), ())),
                        preferred_element_type=jnp.float32)
    z = z + b1_ref[0]
    z = 0.5 * z * (1.0 + lax.erf(z * 0.7071067811865476))
    delta = lax.dot_general(z.astype(jnp.bfloat16),
                            w2_ref[0].astype(jnp.bfloat16),
                            (((1,), (1,)), ((), ())),
                            preferred_element_type=jnp.float32)
    out = x + (delta + b2_ref[0])
    row = lax.broadcasted_iota(jnp.int32, (ROW_BLOCK, 1), 0) + tile * ROW_BLOCK
    mask = jnp.logical_and(row >= off_ref[g], row < off_ref[g + 1])
    out_ref[...] = jnp.where(mask, out, out_ref[...])


def _grouped_delta(hs, W1, b1r, W2, b2r, off, g, tile):
    grid_spec = pltpu.PrefetchScalarGridSpec(
        num_scalar_prefetch=3,
        grid=(GRID_T,),
        in_specs=[
            pl.BlockSpec((ROW_BLOCK, HIDDEN_DIM),
                         lambda t, gr, tr, orf: (tr[t], 0)),
            pl.BlockSpec((1, BOTTLENECK_DIM, HIDDEN_DIM),
                         lambda t, gr, tr, orf: (gr[t], 0, 0)),
            pl.BlockSpec((1, 1, BOTTLENECK_DIM),
                         lambda t, gr, tr, orf: (gr[t], 0, 0)),
            pl.BlockSpec((1, HIDDEN_DIM, BOTTLENECK_DIM),
                         lambda t, gr, tr, orf: (gr[t], 0, 0)),
            pl.BlockSpec((1, 1, HIDDEN_DIM),
                         lambda t, gr, tr, orf: (gr[t], 0, 0)),
        ],
        out_specs=pl.BlockSpec((ROW_BLOCK, HIDDEN_DIM),
                               lambda t, gr, tr, orf: (tr[t], 0)),
    )
    return pl.pallas_call(
        _gmm_body,
        grid_spec=grid_spec,
        out_shape=jax.ShapeDtypeStruct((N_TOKENS, HIDDEN_DIM), jnp.float32),
        compiler_params=pltpu.CompilerParams(
            dimension_semantics=("arbitrary",)),
    )(g, tile, off, hs, W1, b1r, W2, b2r)


def kernel(h, cluster_ids, W1, b1, W2, b2):
    cid = cluster_ids.astype(jnp.int32)
    rank, off, g, tile = _route_metadata(cid)

    hs = _sc_scatter_rows(h, rank)
    b1r = b1.reshape(NUM_CLUSTERS, 1, BOTTLENECK_DIM)
    b2r = b2.reshape(NUM_CLUSTERS, 1, HIDDEN_DIM)

    out_sorted = _grouped_delta(hs, W1, b1r, W2, b2r, off, g, tile)
    return jnp.take(out_sorted, rank, axis=0, mode="clip")
